# baseline (speedup 1.0000x reference)
import functools

import jax
import jax.numpy as jnp
from jax import lax
from jax.experimental import pallas as pl
from jax.experimental.pallas import tpu as pltpu, tpu_sc as plsc

_B = 16384
_IN_F = 256
_OUT_F = 64
_P = 32
_WIDTH = 4.0

_NW = 32
_BPW = _B // _NW
_CH = 64
_NCH = _BPW // _CH
_FB = 16
_NFB = _IN_F // _FB
_NJ = _OUT_F // 16


def _sc_kernel(x_hbm, tab_hbm, out_hbm, xblk, tab, lo_r, t_r, acc):
    wid = lax.axis_index("s") * 2 + lax.axis_index("c")

    def chunk_body(c, _c):
        base = wid * _BPW + c * _CH
        pltpu.sync_copy(x_hbm.at[pl.ds(base, _CH), :], xblk)

        def zero_body(b, _):
            z = jnp.zeros((16,), jnp.float32)
            for j in range(_NJ):
                acc[b, pl.ds(j * 16, 16)] = z
            return _

        lax.fori_loop(0, _CH, zero_body, 0)

        def fblock(fb, _):
            f0 = fb * _FB
            pltpu.sync_copy(tab_hbm.at[pl.ds(fb * _FB * _P, _FB * _P), :], tab)

            def lot_body(b, _a):
                xs = (xblk[b, pl.ds(f0, _FB)] + _WIDTH / 2.0) * ((_P - 1) / _WIDTH)
                lo = jnp.minimum(jnp.maximum(xs, 0.0).astype(jnp.int32), _P - 2)
                lo_r[b, :] = lo
                t_r[b, :] = xs - lo.astype(jnp.float32)
                return _a

            lax.fori_loop(0, _CH, lot_body, 0)

            def row_body(b, _a):
                a = [acc[b, pl.ds(j * 16, 16)] for j in range(_NJ)]
                lo_v = lo_r[b, :]
                t_v = t_r[b, :]
                for f in range(_FB):
                    lo_s = lo_v[f]
                    t_s = t_v[f]
                    row = f * _P + lo_s
                    for j in range(_NJ):
                        rl = tab[row, pl.ds(j * 16, 16)]
                        rh = tab[row + 1, pl.ds(j * 16, 16)]
                        a[j] = a[j] + rl + t_s * (rh - rl)
                for j in range(_NJ):
                    acc[b, pl.ds(j * 16, 16)] = a[j]
                return _a

            lax.fori_loop(0, _CH, row_body, 0)
            return _

        lax.fori_loop(0, _NFB, fblock, 0)
        pltpu.sync_copy(acc, out_hbm.at[pl.ds(base, _CH), :])
        return _c

    lax.fori_loop(0, _NCH, chunk_body, 0)


def kernel(x, kan_weight):
    tab = kan_weight.reshape(_IN_F * _P, _OUT_F)
    mesh = plsc.VectorSubcoreMesh(core_axis_name="c", subcore_axis_name="s")
    f = functools.partial(
        pl.kernel,
        mesh=mesh,
        out_type=jax.ShapeDtypeStruct((_B, _OUT_F), jnp.float32),
        scratch_types=[
            pltpu.VMEM((_CH, _IN_F), jnp.float32),
            pltpu.VMEM((_FB * _P, _OUT_F), jnp.float32),
            pltpu.VMEM((_CH, _FB), jnp.int32),
            pltpu.VMEM((_CH, _FB), jnp.float32),
            pltpu.VMEM((_CH, _OUT_F), jnp.float32),
        ],
    )(_sc_kernel)
    return f(x, tab)


# speedup vs baseline: 7.9550x; 7.9550x over previous
import functools

import jax
import jax.numpy as jnp
from jax import lax
from jax.experimental import pallas as pl
from jax.experimental.pallas import tpu as pltpu, tpu_sc as plsc

_IN_F = 256
_OUT_F = 64
_P = 32
_WIDTH = 4.0

_B_SC = 1024
_NW = 32
_BPW = _B_SC // _NW
_FB = 16
_NFB = _IN_F // _FB
_NJ = _OUT_F // 16


def _sc_body(x_hbm, tab_hbm, out_hbm, xblk, tab, lo_r, t_r, acc):
    wid = lax.axis_index("s") * 2 + lax.axis_index("c")
    base = wid * _BPW
    pltpu.sync_copy(x_hbm.at[pl.ds(base, _BPW), :], xblk)

    def zero_body(b, carry):
        z = jnp.zeros((16,), jnp.float32)
        for j in range(_NJ):
            acc[b, pl.ds(j * 16, 16)] = z
        return carry

    lax.fori_loop(0, _BPW, zero_body, 0)

    def fblock(fb, carry):
        f0 = fb * _FB
        pltpu.sync_copy(tab_hbm.at[pl.ds(fb * _FB * _P, _FB * _P), :], tab)

        def lot_body(b, c2):
            xs = (xblk[b, pl.ds(f0, _FB)] + _WIDTH / 2.0) * ((_P - 1) / _WIDTH)
            lo = jnp.minimum(jnp.maximum(xs, 0.0).astype(jnp.int32), _P - 2)
            lo_r[b, :] = lo
            t_r[b, :] = xs - lo.astype(jnp.float32)
            return c2

        lax.fori_loop(0, _BPW, lot_body, 0)

        def row_body(b, c2):
            a = [acc[b, pl.ds(j * 16, 16)] for j in range(_NJ)]
            lo_v = lo_r[b, :]
            t_v = t_r[b, :]
            for f in range(_FB):
                lo_s = lo_v[f]
                t_s = t_v[f]
                row = f * _P + lo_s
                for j in range(_NJ):
                    rl = tab[row, pl.ds(j * 16, 16)]
                    rh = tab[row + 1, pl.ds(j * 16, 16)]
                    a[j] = a[j] + rl + t_s * (rh - rl)
            for j in range(_NJ):
                acc[b, pl.ds(j * 16, 16)] = a[j]
            return c2

        lax.fori_loop(0, _BPW, row_body, 0)
        return carry

    lax.fori_loop(0, _NFB, fblock, 0)
    pltpu.sync_copy(acc, out_hbm.at[pl.ds(base, _BPW), :])


def _sc_part(x, tab):
    mesh = plsc.VectorSubcoreMesh(core_axis_name="c", subcore_axis_name="s")
    f = functools.partial(
        pl.kernel,
        mesh=mesh,
        out_type=jax.ShapeDtypeStruct((_B_SC, _OUT_F), jnp.float32),
        scratch_types=[
            pltpu.VMEM((_BPW, _IN_F), jnp.float32),
            pltpu.VMEM((_FB * _P, _OUT_F), jnp.float32),
            pltpu.VMEM((_BPW, _FB), jnp.int32),
            pltpu.VMEM((_BPW, _FB), jnp.float32),
            pltpu.VMEM((_BPW, _OUT_F), jnp.float32),
        ],
    )(_sc_body)
    return f(x, tab)


def _tc_body(x_ref, v_ref, b_ref, o_ref):
    p_max = _P - 1
    xs = (x_ref[...] + _WIDTH / 2.0) * (p_max / _WIDTH)

    dot = lambda a, b: jax.lax.dot(a, b, preferred_element_type=jnp.float32)
    acc = b_ref[0:1, :] + dot(xs, v_ref[0])
    for k in range(1, p_max):
        acc += dot(jnp.maximum(xs - float(k), 0.0), v_ref[k])
    o_ref[...] = acc


def _tc_part(x, v, bias, bt):
    b = x.shape[0]
    return pl.pallas_call(
        _tc_body,
        grid=(b // bt,),
        in_specs=[
            pl.BlockSpec((bt, _IN_F), lambda i: (i, 0)),
            pl.BlockSpec((_P - 1, _IN_F, _OUT_F), lambda i: (0, 0, 0)),
            pl.BlockSpec((8, _OUT_F), lambda i: (0, 0)),
        ],
        out_specs=pl.BlockSpec((bt, _OUT_F), lambda i: (i, 0)),
        out_shape=jax.ShapeDtypeStruct((b, _OUT_F), jnp.float32),
    )(x, v, bias)


def kernel(x, kan_weight):
    tab = kan_weight.reshape(_IN_F * _P, _OUT_F)
    wt = jnp.transpose(kan_weight, (1, 0, 2))
    v = jnp.concatenate(
        [
            (wt[1] - wt[0])[None],
            wt[2:] - 2.0 * wt[1:-1] + wt[:-2],
        ],
        axis=0,
    )
    bias = jnp.broadcast_to(jnp.sum(wt[0], axis=0)[None, :], (8, _OUT_F))
    out_sc = _sc_part(x[:_B_SC], tab)
    out_tc = _tc_part(x[_B_SC:], v, bias, bt=1024)
    return jnp.concatenate([out_sc, out_tc], axis=0)


# speedup vs baseline: 9.0265x; 1.1347x over previous
import functools

import jax
import jax.numpy as jnp
from jax import lax
from jax.experimental import pallas as pl
from jax.experimental.pallas import tpu as pltpu, tpu_sc as plsc

_IN_F = 256
_OUT_F = 64
_P = 32
_WIDTH = 4.0

_B_SC = 512
_NW = 32
_BPW = _B_SC // _NW
_FB = 16
_NFB = _IN_F // _FB
_NJ = _OUT_F // 16


def _sc_body(x_hbm, tab_hbm, out_hbm, xblk, tab, lo_r, t_r, acc):
    wid = lax.axis_index("s") * 2 + lax.axis_index("c")
    base = wid * _BPW
    pltpu.sync_copy(x_hbm.at[pl.ds(base, _BPW), :], xblk)

    def zero_body(b, carry):
        z = jnp.zeros((16,), jnp.float32)
        for j in range(_NJ):
            acc[b, pl.ds(j * 16, 16)] = z
        return carry

    lax.fori_loop(0, _BPW, zero_body, 0)

    def fblock(fb, carry):
        f0 = fb * _FB
        pltpu.sync_copy(tab_hbm.at[pl.ds(fb * _FB * _P, _FB * _P), :], tab)

        def lot_body(b, c2):
            xs = (xblk[b, pl.ds(f0, _FB)] + _WIDTH / 2.0) * ((_P - 1) / _WIDTH)
            lo = jnp.minimum(jnp.maximum(xs, 0.0).astype(jnp.int32), _P - 2)
            lo_r[b, :] = lo
            t_r[b, :] = xs - lo.astype(jnp.float32)
            return c2

        lax.fori_loop(0, _BPW, lot_body, 0)

        def row_body(b, c2):
            a = [acc[b, pl.ds(j * 16, 16)] for j in range(_NJ)]
            lo_v = lo_r[b, :]
            t_v = t_r[b, :]
            for f in range(_FB):
                lo_s = lo_v[f]
                t_s = t_v[f]
                row = f * _P + lo_s
                for j in range(_NJ):
                    rl = tab[row, pl.ds(j * 16, 16)]
                    rh = tab[row + 1, pl.ds(j * 16, 16)]
                    a[j] = a[j] + rl + t_s * (rh - rl)
            for j in range(_NJ):
                acc[b, pl.ds(j * 16, 16)] = a[j]
            return c2

        lax.fori_loop(0, _BPW, row_body, 0)
        return carry

    lax.fori_loop(0, _NFB, fblock, 0)
    pltpu.sync_copy(acc, out_hbm.at[pl.ds(base, _BPW), :])


def _sc_part(x, tab):
    mesh = plsc.VectorSubcoreMesh(core_axis_name="c", subcore_axis_name="s")
    f = functools.partial(
        pl.kernel,
        mesh=mesh,
        out_type=jax.ShapeDtypeStruct((_B_SC, _OUT_F), jnp.float32),
        scratch_types=[
            pltpu.VMEM((_BPW, _IN_F), jnp.float32),
            pltpu.VMEM((_FB * _P, _OUT_F), jnp.float32),
            pltpu.VMEM((_BPW, _FB), jnp.int32),
            pltpu.VMEM((_BPW, _FB), jnp.float32),
            pltpu.VMEM((_BPW, _OUT_F), jnp.float32),
        ],
    )(_sc_body)
    return f(x, tab)


def _tc_body(x_ref, v_ref, b_ref, o_ref):
    p_max = _P - 1
    xs = (x_ref[...] + _WIDTH / 2.0) * (p_max / _WIDTH)

    dot = lambda a, b: jax.lax.dot(a, b, preferred_element_type=jnp.float32)
    acc = b_ref[0:1, :] + dot(xs, v_ref[0])
    for k in range(1, p_max):
        acc += dot(jnp.maximum(xs - float(k), 0.0), v_ref[k])
    o_ref[...] = acc


def _tc_part(x, v, bias, bt):
    b = x.shape[0]
    return pl.pallas_call(
        _tc_body,
        grid=(b // bt,),
        in_specs=[
            pl.BlockSpec((bt, _IN_F), lambda i: (i, 0)),
            pl.BlockSpec((_P - 1, _IN_F, _OUT_F), lambda i: (0, 0, 0)),
            pl.BlockSpec((8, _OUT_F), lambda i: (0, 0)),
        ],
        out_specs=pl.BlockSpec((bt, _OUT_F), lambda i: (i, 0)),
        out_shape=jax.ShapeDtypeStruct((b, _OUT_F), jnp.float32),
    )(x, v, bias)


def kernel(x, kan_weight):
    tab = kan_weight.reshape(_IN_F * _P, _OUT_F)
    wt = jnp.transpose(kan_weight, (1, 0, 2))
    v = jnp.concatenate(
        [
            (wt[1] - wt[0])[None],
            wt[2:] - 2.0 * wt[1:-1] + wt[:-2],
        ],
        axis=0,
    )
    bias = jnp.broadcast_to(jnp.sum(wt[0], axis=0)[None, :], (8, _OUT_F))
    out_sc = _sc_part(x[:_B_SC], tab)
    out_tc = _tc_part(x[_B_SC:], v, bias, bt=512)
    return jnp.concatenate([out_sc, out_tc], axis=0)
